# Initial kernel scaffold; baseline (speedup 1.0000x reference)
import jax
import jax.numpy as jnp
from jax.experimental import pallas as pl


def kernel(ub, lb, W_upper, b_upper, W_lower, b_lower, alpha, input_ub, input_lb):
    raise NotImplementedError("write your pallas kernel here")


# speedup vs baseline: 1.0336x; 1.0336x over previous
import functools

import jax
import jax.numpy as jnp
from jax.experimental import pallas as pl

N = 4096
D = 2048
B = 128


def _relu_kernel(ub_ref, lb_ref, Wu_in_ref, bu_in_ref, Wl_in_ref, bl_in_ref,
                 alpha_ref,
                 new_ub_ref, new_lb_ref, Wu_ref, bu_ref, Wl_ref, bl_ref,
                 Wu2_ref, bu2_ref, Wl2_ref, bl2_ref):
    i = pl.program_id(0)
    ub = ub_ref[:]
    lb = lb_ref[:]
    alpha = alpha_ref[:]
    bu_in = bu_in_ref[:]
    bl_in = bl_in_ref[:]

    neg = ub <= 0.0
    pos = lb >= 0.0
    cross = jnp.logical_not(jnp.logical_or(neg, pos))
    alpha_c = jnp.clip(alpha, 0.0, 1.0)
    denom = jnp.where(cross, ub - lb, 1.0)
    a = jnp.where(cross, ub / denom, 0.0)
    b = -lb * a

    new_ub_ref[:] = jnp.where(neg, 0.0, ub)
    new_lb_ref[:] = jnp.where(pos, lb, jnp.where(cross, alpha_c * lb, 0.0))
    bu_ref[:] = jnp.where(pos, bu_in, jnp.where(cross, bu_in + b, 0.0))
    bl_ref[:] = jnp.where(pos, bl_in, jnp.where(cross, bu_in, 0.0))
    bu2_ref[:] = jnp.where(cross, b, 0.0)
    bl2_ref[:] = jnp.zeros_like(b)

    u_scale = jnp.where(pos, 1.0, a)
    l_scale = jnp.where(pos, 1.0, jnp.where(cross, alpha_c, 0.0))
    Wu_ref[:, :] = u_scale[:, None] * Wu_in_ref[:, :]
    Wl_ref[:, :] = l_scale[:, None] * Wl_in_ref[:, :]

    du = jnp.where(cross, a, 1.0)
    dl = jnp.where(cross, alpha_c, 1.0)
    rows = jax.lax.broadcasted_iota(jnp.int32, (B, N), 0) + i * B
    cols = jax.lax.broadcasted_iota(jnp.int32, (B, N), 1)
    on_diag = rows == cols
    Wu2_ref[:, :] = jnp.where(on_diag, du[:, None], 0.0)
    Wl2_ref[:, :] = jnp.where(on_diag, dl[:, None], 0.0)


@functools.partial(jax.jit, static_argnames=())
def kernel(ub, lb, W_upper, b_upper, W_lower, b_lower, alpha, input_ub, input_lb):
    del input_ub, input_lb
    grid = (N // B,)
    vec_spec = pl.BlockSpec((B,), lambda i: (i,))
    mat_spec = pl.BlockSpec((B, D), lambda i: (i, 0))
    diag_spec = pl.BlockSpec((B, N), lambda i: (i, 0))
    f32 = jnp.float32
    out_shapes = (
        jax.ShapeDtypeStruct((N,), f32),
        jax.ShapeDtypeStruct((N,), f32),
        jax.ShapeDtypeStruct((N, D), f32),
        jax.ShapeDtypeStruct((N,), f32),
        jax.ShapeDtypeStruct((N, D), f32),
        jax.ShapeDtypeStruct((N,), f32),
        jax.ShapeDtypeStruct((N, N), f32),
        jax.ShapeDtypeStruct((N,), f32),
        jax.ShapeDtypeStruct((N, N), f32),
        jax.ShapeDtypeStruct((N,), f32),
    )
    out_specs = (vec_spec, vec_spec, mat_spec, vec_spec, mat_spec, vec_spec,
                 diag_spec, vec_spec, diag_spec, vec_spec)
    in_specs = (vec_spec, vec_spec, mat_spec, vec_spec, mat_spec, vec_spec,
                vec_spec)
    return pl.pallas_call(
        _relu_kernel,
        grid=grid,
        in_specs=in_specs,
        out_specs=out_specs,
        out_shape=out_shapes,
    )(ub, lb, W_upper, b_upper, W_lower, b_lower, alpha)


# speedup vs baseline: 1.0763x; 1.0414x over previous
import functools

import jax
import jax.numpy as jnp
from jax.experimental import pallas as pl

N = 4096
D = 2048
B = 256


def _relu_kernel(ub_ref, lb_ref, Wu_in_ref, bu_in_ref, Wl_in_ref, bl_in_ref,
                 alpha_ref,
                 new_ub_ref, new_lb_ref, Wu_ref, bu_ref, Wl_ref, bl_ref,
                 Wu2_ref, bu2_ref, Wl2_ref, bl2_ref):
    i = pl.program_id(0)
    ub = ub_ref[:]
    lb = lb_ref[:]
    alpha = alpha_ref[:]
    bu_in = bu_in_ref[:]
    bl_in = bl_in_ref[:]

    neg = ub <= 0.0
    pos = lb >= 0.0
    cross = jnp.logical_not(jnp.logical_or(neg, pos))
    alpha_c = jnp.clip(alpha, 0.0, 1.0)
    denom = jnp.where(cross, ub - lb, 1.0)
    a = jnp.where(cross, ub / denom, 0.0)
    b = -lb * a

    new_ub_ref[:] = jnp.where(neg, 0.0, ub)
    new_lb_ref[:] = jnp.where(pos, lb, jnp.where(cross, alpha_c * lb, 0.0))
    bu_ref[:] = jnp.where(pos, bu_in, jnp.where(cross, bu_in + b, 0.0))
    bl_ref[:] = jnp.where(pos, bl_in, jnp.where(cross, bu_in, 0.0))
    bu2_ref[:] = jnp.where(cross, b, 0.0)
    bl2_ref[:] = jnp.zeros_like(b)

    u_scale = jnp.where(pos, 1.0, a)
    l_scale = jnp.where(pos, 1.0, jnp.where(cross, alpha_c, 0.0))
    Wu_ref[:, :] = u_scale[:, None] * Wu_in_ref[:, :]
    Wl_ref[:, :] = l_scale[:, None] * Wl_in_ref[:, :]

    du = jnp.where(cross, a, 1.0)
    dl = jnp.where(cross, alpha_c, 1.0)
    rows = jax.lax.broadcasted_iota(jnp.int32, (B, N), 0) + i * B
    cols = jax.lax.broadcasted_iota(jnp.int32, (B, N), 1)
    on_diag = rows == cols
    Wu2_ref[:, :] = jnp.where(on_diag, du[:, None], 0.0)
    Wl2_ref[:, :] = jnp.where(on_diag, dl[:, None], 0.0)


@functools.partial(jax.jit, static_argnames=())
def kernel(ub, lb, W_upper, b_upper, W_lower, b_lower, alpha, input_ub, input_lb):
    del input_ub, input_lb
    grid = (N // B,)
    vec_spec = pl.BlockSpec((B,), lambda i: (i,))
    mat_spec = pl.BlockSpec((B, D), lambda i: (i, 0))
    diag_spec = pl.BlockSpec((B, N), lambda i: (i, 0))
    f32 = jnp.float32
    out_shapes = (
        jax.ShapeDtypeStruct((N,), f32),
        jax.ShapeDtypeStruct((N,), f32),
        jax.ShapeDtypeStruct((N, D), f32),
        jax.ShapeDtypeStruct((N,), f32),
        jax.ShapeDtypeStruct((N, D), f32),
        jax.ShapeDtypeStruct((N,), f32),
        jax.ShapeDtypeStruct((N, N), f32),
        jax.ShapeDtypeStruct((N,), f32),
        jax.ShapeDtypeStruct((N, N), f32),
        jax.ShapeDtypeStruct((N,), f32),
    )
    out_specs = (vec_spec, vec_spec, mat_spec, vec_spec, mat_spec, vec_spec,
                 diag_spec, vec_spec, diag_spec, vec_spec)
    in_specs = (vec_spec, vec_spec, mat_spec, vec_spec, mat_spec, vec_spec,
                vec_spec)
    return pl.pallas_call(
        _relu_kernel,
        grid=grid,
        in_specs=in_specs,
        out_specs=out_specs,
        out_shape=out_shapes,
    )(ub, lb, W_upper, b_upper, W_lower, b_lower, alpha)
